# baseline (speedup 1.0000x reference)
import functools

import jax
import jax.numpy as jnp
from jax import lax
from jax.experimental import pallas as pl
from jax.experimental.pallas import tpu as pltpu
from jax.experimental.pallas import tpu_sc as plsc

N_NODES = 10000
N_EDGES = 320000
NODE_DIM = 128
EDGE_DIM = 16
PAIR_DIM = 2 * NODE_DIM
EPS = 1e-5

NC = 2
NS = 16
NW = NC * NS
E_PER_W = N_EDGES // NW
CHUNK = 80
N_CHUNKS = E_PER_W // CHUNK

EDGE_BLK = 16000
N_EDGE_BLKS = N_EDGES // EDGE_BLK
D_BLK = 10000
N_D_BLKS = N_EDGES // D_BLK
NODE_BLK = 2000
N_NODE_BLKS = N_NODES // NODE_BLK

@functools.cache
def _sc_mesh():
    return plsc.VectorSubcoreMesh(core_axis_name="c", subcore_axis_name="s",
                                  num_cores=NC, num_subcores=NS)


def _pack_bf16_pair(core_f32, filt_f32):
    cb = lax.bitcast_convert_type(core_f32.astype(jnp.bfloat16),
                                  jnp.uint16).astype(jnp.int32)
    fb = lax.bitcast_convert_type(filt_f32.astype(jnp.bfloat16),
                                  jnp.uint16).astype(jnp.int32)
    return cb | (fb << 16)


def _unpack_bf16_pair(packed_i32):
    core = lax.bitcast_convert_type(packed_i32 << 16, jnp.float32)
    filt = lax.bitcast_convert_type((packed_i32 >> 16) << 16, jnp.float32)
    return core, filt


def _proj_body(a_ref, wd_ref, ws_ref, pd_ref, ps_ref):
    a = a_ref[...]
    pd = jnp.dot(a, wd_ref[...], preferred_element_type=jnp.float32)
    ps = jnp.dot(a, ws_ref[...], preferred_element_type=jnp.float32)
    pd_ref[...] = _pack_bf16_pair(pd[:, :NODE_DIM], pd[:, NODE_DIM:])
    ps_ref[...] = _pack_bf16_pair(ps[:, :NODE_DIM], ps[:, NODE_DIM:])


def _project(atom, Wd, Ws):
    return pl.pallas_call(
        _proj_body,
        grid=(N_NODE_BLKS,),
        in_specs=[
            pl.BlockSpec((NODE_BLK, NODE_DIM), lambda i: (i, 0)),
            pl.BlockSpec((NODE_DIM, PAIR_DIM), lambda i: (0, 0)),
            pl.BlockSpec((NODE_DIM, PAIR_DIM), lambda i: (0, 0)),
        ],
        out_specs=[
            pl.BlockSpec((NODE_BLK, NODE_DIM), lambda i: (i, 0)),
            pl.BlockSpec((NODE_BLK, NODE_DIM), lambda i: (i, 0)),
        ],
        out_shape=[
            jax.ShapeDtypeStruct((N_NODES, NODE_DIM), jnp.int32),
            jax.ShapeDtypeStruct((N_NODES, NODE_DIM), jnp.int32),
        ],
    )(atom, Wd, Ws)


SLOTS = 5
N_GROUPS = N_CHUNKS // SLOTS
SC_SLOTS = 4
N_SC_GROUPS = (N_CHUNKS - 1) // SC_SLOTS


@functools.cache
def _gather_kernel_fn():
    @functools.partial(
        pl.kernel,
        out_type=[
            jax.ShapeDtypeStruct((N_EDGES, NODE_DIM), jnp.int32),
            jax.ShapeDtypeStruct((N_EDGES, NODE_DIM), jnp.int32),
        ],
        mesh=_sc_mesh(),
        scratch_types=[
            pltpu.VMEM((E_PER_W,), jnp.int32),
            pltpu.VMEM((E_PER_W,), jnp.int32),
            pltpu.VMEM((SLOTS, CHUNK, NODE_DIM), jnp.int32),
            pltpu.VMEM((SLOTS, CHUNK, NODE_DIM), jnp.int32),
            pltpu.SemaphoreType.DMA,
            pltpu.SemaphoreType.DMA,
        ],
    )
    def _gather_kernel(pd_hbm, ps_hbm, dsti_hbm, srci_hbm, gd_hbm, gs_hbm,
                       di_v, si_v, rd_v, rs_v, semg, semw):
        wid = lax.axis_index("s") * NC + lax.axis_index("c")
        base = wid * E_PER_W

        pltpu.sync_copy(dsti_hbm.at[pl.ds(base, E_PER_W)], di_v)
        pltpu.sync_copy(srci_hbm.at[pl.ds(base, E_PER_W)], si_v)

        def drain_writes():
            for t in range(SLOTS):
                pltpu.make_async_copy(
                    rd_v.at[t], gd_hbm.at[pl.ds(base, CHUNK)], semw).wait()
                pltpu.make_async_copy(
                    rs_v.at[t], gs_hbm.at[pl.ds(base, CHUNK)], semw).wait()

        def do_group(g, drain_prev):
            if drain_prev:
                drain_writes()
            descs = []
            for t in range(SLOTS):
                loc = (g * SLOTS + t) * CHUNK
                descs.append(pltpu.async_copy(
                    pd_hbm.at[di_v.at[pl.ds(loc, CHUNK)]], rd_v.at[t], semg))
                descs.append(pltpu.async_copy(
                    ps_hbm.at[si_v.at[pl.ds(loc, CHUNK)]], rs_v.at[t], semg))
            for d in descs:
                d.wait()
            for t in range(SLOTS):
                off = base + (g * SLOTS + t) * CHUNK
                pltpu.async_copy(rd_v.at[t], gd_hbm.at[pl.ds(off, CHUNK)], semw)
                pltpu.async_copy(rs_v.at[t], gs_hbm.at[pl.ds(off, CHUNK)], semw)

        do_group(0, False)

        def body(g, carry):
            do_group(g, True)
            return carry

        lax.fori_loop(1, N_GROUPS, body, 0)
        drain_writes()

    return _gather_kernel


def _stats_body(gd_ref, gs_ref, ef_ref, we_ref, out_ref, acc_ref):
    i = pl.program_id(0)

    @pl.when(i == 0)
    def _init():
        acc_ref[...] = jnp.zeros_like(acc_ref)

    dc, df = _unpack_bf16_pair(gd_ref[...])
    sc_, sf = _unpack_bf16_pair(gs_ref[...])
    y = (jnp.concatenate([dc + sc_, df + sf], axis=1)
         + jnp.dot(ef_ref[...], we_ref[...],
                   preferred_element_type=jnp.float32))
    acc_ref[0:1, :] += jnp.sum(y, axis=0, keepdims=True)
    acc_ref[1:2, :] += jnp.sum(y * y, axis=0, keepdims=True)

    @pl.when(i == pl.num_programs(0) - 1)
    def _fin():
        out_ref[...] = acc_ref[...]


def _stats(gd, gs, ef, We):
    return pl.pallas_call(
        _stats_body,
        grid=(N_EDGE_BLKS,),
        in_specs=[
            pl.BlockSpec((EDGE_BLK, NODE_DIM), lambda i: (i, 0)),
            pl.BlockSpec((EDGE_BLK, NODE_DIM), lambda i: (i, 0)),
            pl.BlockSpec((EDGE_BLK, EDGE_DIM), lambda i: (i, 0)),
            pl.BlockSpec((EDGE_DIM, PAIR_DIM), lambda i: (0, 0)),
        ],
        out_specs=pl.BlockSpec((2, PAIR_DIM), lambda i: (0, 0)),
        out_shape=jax.ShapeDtypeStruct((2, PAIR_DIM), jnp.float32),
        scratch_shapes=[pltpu.VMEM((2, PAIR_DIM), jnp.float32)],
    )(gd, gs, ef, We)


def _apply_body(s_ref, b_ref, gam_ref, bet_ref, gd_ref, gs_ref, ef_ref, we_ref,
                msg_ref):
    inv_e = 1.0 / N_EDGES
    m0 = s_ref[0:1, :] * inv_e
    m2 = s_ref[1:2, :] * inv_e
    var = m2 - m0 * m0
    rstd = lax.rsqrt(var + EPS)
    scale = gam_ref[...] * rstd
    shift = bet_ref[...] - (m0 + b_ref[...]) * scale
    dc, df = _unpack_bf16_pair(gd_ref[...])
    sc_, sf = _unpack_bf16_pair(gs_ref[...])
    y = (jnp.concatenate([dc + sc_, df + sf], axis=1)
         + jnp.dot(ef_ref[...], we_ref[...],
                   preferred_element_type=jnp.float32))
    yn = (y + b_ref[...]) * scale + shift
    sig = yn[:, :NODE_DIM]
    gat = yn[:, NODE_DIM:]
    signal = jnp.maximum(sig, 0.0) + jnp.log(1.0 + jnp.exp(-jnp.abs(sig)))
    gate = 1.0 / (1.0 + jnp.exp(-gat))
    msg_ref[...] = gate * signal


def _apply(stats, bias, gam, bet, gd, gs, ef, We):
    return pl.pallas_call(
        _apply_body,
        grid=(N_D_BLKS,),
        in_specs=[
            pl.BlockSpec((2, PAIR_DIM), lambda i: (0, 0)),
            pl.BlockSpec((1, PAIR_DIM), lambda i: (0, 0)),
            pl.BlockSpec((1, PAIR_DIM), lambda i: (0, 0)),
            pl.BlockSpec((1, PAIR_DIM), lambda i: (0, 0)),
            pl.BlockSpec((D_BLK, NODE_DIM), lambda i: (i, 0)),
            pl.BlockSpec((D_BLK, NODE_DIM), lambda i: (i, 0)),
            pl.BlockSpec((D_BLK, EDGE_DIM), lambda i: (i, 0)),
            pl.BlockSpec((EDGE_DIM, PAIR_DIM), lambda i: (0, 0)),
        ],
        out_specs=pl.BlockSpec((D_BLK, NODE_DIM), lambda i: (i, 0)),
        out_shape=jax.ShapeDtypeStruct((N_EDGES, NODE_DIM), jnp.float32),
    )(stats, bias, gam, bet, gd, gs, ef, We)


_STRIPE = 640
_STRIPE_LAST = N_NODES - 15 * _STRIPE


@functools.cache
def _scatter_kernel_fn():
    @functools.partial(
        pl.kernel,
        out_type=jax.ShapeDtypeStruct((NC, N_NODES, NODE_DIM), jnp.float32),
        mesh=_sc_mesh(),
        scratch_types=[
            [pltpu.VMEM((CHUNK,), jnp.int32) for _ in range(SC_SLOTS)],
            pltpu.VMEM((SC_SLOTS, CHUNK, NODE_DIM), jnp.float32),
            pltpu.VMEM_SHARED((N_NODES, NODE_DIM), jnp.float32),
            pltpu.SemaphoreType.DMA,
            pltpu.SemaphoreType.DMA,
        ],
    )
    def _scatter_kernel(msg_hbm, dsti_hbm, zer_hbm, out_hbm, di_v, rows_v,
                        acc_sh, semr, sema):
        c = lax.axis_index("c")
        s = lax.axis_index("s")
        wid = s * NC + c
        base = wid * E_PER_W

        @pl.when(s < NS - 1)
        def _z0():
            pltpu.sync_copy(zer_hbm.at[pl.ds(s * _STRIPE, _STRIPE)],
                            acc_sh.at[pl.ds(s * _STRIPE, _STRIPE)])

        @pl.when(s == NS - 1)
        def _z1():
            pltpu.sync_copy(zer_hbm.at[pl.ds(15 * _STRIPE, _STRIPE_LAST)],
                            acc_sh.at[pl.ds(15 * _STRIPE, _STRIPE_LAST)])

        plsc.subcore_barrier()

        def drain_adds():
            for t in range(SC_SLOTS):
                pltpu.make_async_copy(
                    rows_v.at[t], acc_sh.at[di_v[t]], sema).wait()

        def do_group(g, drain_prev):
            if drain_prev:
                drain_adds()
            descs = []
            for t in range(SC_SLOTS):
                off = base + (g * SC_SLOTS + t) * CHUNK
                descs.append(pltpu.async_copy(
                    dsti_hbm.at[pl.ds(off, CHUNK)], di_v[t], semr))
                descs.append(pltpu.async_copy(
                    msg_hbm.at[pl.ds(off, CHUNK)], rows_v.at[t], semr))
            for d in descs:
                d.wait()
            for t in range(SC_SLOTS):
                pltpu.async_copy(rows_v.at[t], acc_sh.at[di_v[t]], sema,
                                 add=True)

        do_group(0, False)

        def body(g, carry):
            do_group(g, True)
            return carry

        lax.fori_loop(1, N_SC_GROUPS, body, 0)
        drain_adds()

        tail_off = base + (N_CHUNKS - 1) * CHUNK
        pltpu.sync_copy(dsti_hbm.at[pl.ds(tail_off, CHUNK)], di_v[0])
        pltpu.sync_copy(msg_hbm.at[pl.ds(tail_off, CHUNK)], rows_v.at[0])
        pltpu.sync_copy(rows_v.at[0], acc_sh.at[di_v[0]], add=True)
        plsc.subcore_barrier()

        @pl.when(s < NS - 1)
        def _w0():
            pltpu.sync_copy(acc_sh.at[pl.ds(s * _STRIPE, _STRIPE)],
                            out_hbm.at[c].at[pl.ds(s * _STRIPE, _STRIPE)])

        @pl.when(s == NS - 1)
        def _w1():
            pltpu.sync_copy(acc_sh.at[pl.ds(15 * _STRIPE, _STRIPE_LAST)],
                            out_hbm.at[c].at[pl.ds(15 * _STRIPE, _STRIPE_LAST)])

    return _scatter_kernel


def _final_body(a_ref, p0_ref, p1_ref, o_ref):
    o_ref[...] = a_ref[...] + p0_ref[0] + p1_ref[0]


def _final(atom, partials):
    return pl.pallas_call(
        _final_body,
        grid=(N_NODE_BLKS,),
        in_specs=[
            pl.BlockSpec((NODE_BLK, NODE_DIM), lambda i: (i, 0)),
            pl.BlockSpec((1, NODE_BLK, NODE_DIM), lambda i: (0, i, 0)),
            pl.BlockSpec((1, NODE_BLK, NODE_DIM), lambda i: (1, i, 0)),
        ],
        out_specs=pl.BlockSpec((NODE_BLK, NODE_DIM), lambda i: (i, 0)),
        out_shape=jax.ShapeDtypeStruct((N_NODES, NODE_DIM), jnp.float32),
    )(atom, partials, partials)


def kernel(atom_features, edge_features, edge_indices, W_filter, b_filter,
           gamma_filter, beta_filter, W_core, b_core, gamma_core, beta_core):
    src_idx = edge_indices[:, 0]
    dst_idx = edge_indices[:, 1]

    Wd = jnp.concatenate([W_core[:NODE_DIM], W_filter[:NODE_DIM]], axis=1)
    Ws = jnp.concatenate([W_core[NODE_DIM:2 * NODE_DIM],
                          W_filter[NODE_DIM:2 * NODE_DIM]], axis=1)
    We = jnp.concatenate([W_core[2 * NODE_DIM:], W_filter[2 * NODE_DIM:]],
                         axis=1)
    bias = jnp.concatenate([b_core, b_filter])[None, :]
    gam = jnp.concatenate([gamma_core, gamma_filter])[None, :]
    bet = jnp.concatenate([beta_core, beta_filter])[None, :]

    pd, ps = _project(atom_features, Wd, Ws)
    gd, gs = _gather_kernel_fn()(pd, ps, dst_idx, src_idx)
    stats = _stats(gd, gs, edge_features, We)
    msg = _apply(stats, bias, gam, bet, gd, gs, edge_features, We)
    zeros = jnp.zeros((N_NODES, NODE_DIM), jnp.float32)
    partials = _scatter_kernel_fn()(msg, dst_idx, zeros)
    return _final(atom_features, partials)


# speedup vs baseline: 1.0123x; 1.0123x over previous
import functools

import jax
import jax.numpy as jnp
from jax import lax
from jax.experimental import pallas as pl
from jax.experimental.pallas import tpu as pltpu
from jax.experimental.pallas import tpu_sc as plsc

N_NODES = 10000
N_EDGES = 320000
NODE_DIM = 128
EDGE_DIM = 16
PAIR_DIM = 2 * NODE_DIM
EPS = 1e-5

HALF_E = N_EDGES // 2

NC = 2
NS = 16
NW = NC * NS
E_PER_W = HALF_E // NW
CHUNK = 40
N_CHUNKS = E_PER_W // CHUNK

SLOTS = 5
N_GROUPS = N_CHUNKS // SLOTS
SC_SLOTS = 4
N_SC_GROUPS = (N_CHUNKS - 1) // SC_SLOTS

EDGE_BLK = 8000
N_EDGE_BLKS = HALF_E // EDGE_BLK
NODE_BLK = 2000
N_NODE_BLKS = N_NODES // NODE_BLK


@functools.cache
def _sc_mesh():
    return plsc.VectorSubcoreMesh(core_axis_name="c", subcore_axis_name="s",
                                  num_cores=NC, num_subcores=NS)


def _pack_bf16_pair(core_f32, filt_f32):
    cb = lax.bitcast_convert_type(core_f32.astype(jnp.bfloat16),
                                  jnp.uint16).astype(jnp.int32)
    fb = lax.bitcast_convert_type(filt_f32.astype(jnp.bfloat16),
                                  jnp.uint16).astype(jnp.int32)
    return cb | (fb << 16)


def _unpack_bf16_pair(packed_i32):
    core = lax.bitcast_convert_type(packed_i32 << 16, jnp.float32)
    filt = lax.bitcast_convert_type((packed_i32 >> 16) << 16, jnp.float32)
    return core, filt


def _proj_body(a_ref, wd_ref, ws_ref, pd_ref, ps_ref):
    a = a_ref[...]
    pd = jnp.dot(a, wd_ref[...], preferred_element_type=jnp.float32)
    ps = jnp.dot(a, ws_ref[...], preferred_element_type=jnp.float32)
    pd_ref[...] = _pack_bf16_pair(pd[:, :NODE_DIM], pd[:, NODE_DIM:])
    ps_ref[...] = _pack_bf16_pair(ps[:, :NODE_DIM], ps[:, NODE_DIM:])


def _project(atom, Wd, Ws):
    return pl.pallas_call(
        _proj_body,
        grid=(N_NODE_BLKS,),
        in_specs=[
            pl.BlockSpec((NODE_BLK, NODE_DIM), lambda i: (i, 0)),
            pl.BlockSpec((NODE_DIM, PAIR_DIM), lambda i: (0, 0)),
            pl.BlockSpec((NODE_DIM, PAIR_DIM), lambda i: (0, 0)),
        ],
        out_specs=[
            pl.BlockSpec((NODE_BLK, NODE_DIM), lambda i: (i, 0)),
            pl.BlockSpec((NODE_BLK, NODE_DIM), lambda i: (i, 0)),
        ],
        out_shape=[
            jax.ShapeDtypeStruct((N_NODES, NODE_DIM), jnp.int32),
            jax.ShapeDtypeStruct((N_NODES, NODE_DIM), jnp.int32),
        ],
    )(atom, Wd, Ws)


@functools.cache
def _gather_kernel_fn():
    @functools.partial(
        pl.kernel,
        out_type=[
            jax.ShapeDtypeStruct((HALF_E, NODE_DIM), jnp.int32),
            jax.ShapeDtypeStruct((HALF_E, NODE_DIM), jnp.int32),
        ],
        mesh=_sc_mesh(),
        scratch_types=[
            pltpu.VMEM((E_PER_W,), jnp.int32),
            pltpu.VMEM((E_PER_W,), jnp.int32),
            pltpu.VMEM((SLOTS, CHUNK, NODE_DIM), jnp.int32),
            pltpu.VMEM((SLOTS, CHUNK, NODE_DIM), jnp.int32),
            pltpu.SemaphoreType.DMA,
            pltpu.SemaphoreType.DMA,
        ],
    )
    def _gather_kernel(pd_hbm, ps_hbm, dsti_hbm, srci_hbm, gd_hbm, gs_hbm,
                       di_v, si_v, rd_v, rs_v, semg, semw):
        wid = lax.axis_index("s") * NC + lax.axis_index("c")
        base = wid * E_PER_W

        pltpu.sync_copy(dsti_hbm.at[pl.ds(base, E_PER_W)], di_v)
        pltpu.sync_copy(srci_hbm.at[pl.ds(base, E_PER_W)], si_v)

        def drain_writes():
            for t in range(SLOTS):
                pltpu.make_async_copy(
                    rd_v.at[t], gd_hbm.at[pl.ds(base, CHUNK)], semw).wait()
                pltpu.make_async_copy(
                    rs_v.at[t], gs_hbm.at[pl.ds(base, CHUNK)], semw).wait()

        def do_group(g, drain_prev):
            if drain_prev:
                drain_writes()
            descs = []
            for t in range(SLOTS):
                loc = (g * SLOTS + t) * CHUNK
                descs.append(pltpu.async_copy(
                    pd_hbm.at[di_v.at[pl.ds(loc, CHUNK)]], rd_v.at[t], semg))
                descs.append(pltpu.async_copy(
                    ps_hbm.at[si_v.at[pl.ds(loc, CHUNK)]], rs_v.at[t], semg))
            for d in descs:
                d.wait()
            for t in range(SLOTS):
                off = base + (g * SLOTS + t) * CHUNK
                pltpu.async_copy(rd_v.at[t], gd_hbm.at[pl.ds(off, CHUNK)], semw)
                pltpu.async_copy(rs_v.at[t], gs_hbm.at[pl.ds(off, CHUNK)], semw)

        do_group(0, False)

        def body(g, carry):
            do_group(g, True)
            return carry

        lax.fori_loop(1, N_GROUPS, body, 0)
        drain_writes()

    return _gather_kernel


def _stats_body(gd_ref, gs_ref, ef_ref, we_ref, out_ref, acc_ref):
    i = pl.program_id(0)

    @pl.when(i == 0)
    def _init():
        acc_ref[...] = jnp.zeros_like(acc_ref)

    dc, df = _unpack_bf16_pair(gd_ref[...])
    sc_, sf = _unpack_bf16_pair(gs_ref[...])
    y = (jnp.concatenate([dc + sc_, df + sf], axis=1)
         + jnp.dot(ef_ref[...], we_ref[...],
                   preferred_element_type=jnp.float32))
    acc_ref[0:1, :] += jnp.sum(y, axis=0, keepdims=True)
    acc_ref[1:2, :] += jnp.sum(y * y, axis=0, keepdims=True)

    @pl.when(i == pl.num_programs(0) - 1)
    def _fin():
        out_ref[...] = acc_ref[...]


def _stats(gd, gs, ef, We):
    return pl.pallas_call(
        _stats_body,
        grid=(N_EDGE_BLKS,),
        in_specs=[
            pl.BlockSpec((EDGE_BLK, NODE_DIM), lambda i: (i, 0)),
            pl.BlockSpec((EDGE_BLK, NODE_DIM), lambda i: (i, 0)),
            pl.BlockSpec((EDGE_BLK, EDGE_DIM), lambda i: (i, 0)),
            pl.BlockSpec((EDGE_DIM, PAIR_DIM), lambda i: (0, 0)),
        ],
        out_specs=pl.BlockSpec((2, PAIR_DIM), lambda i: (0, 0)),
        out_shape=jax.ShapeDtypeStruct((2, PAIR_DIM), jnp.float32),
        scratch_shapes=[pltpu.VMEM((2, PAIR_DIM), jnp.float32)],
    )(gd, gs, ef, We)


def _apply_body(s0_ref, s1_ref, b_ref, gam_ref, bet_ref, gd_ref, gs_ref,
                ef_ref, we_ref, msg_ref):
    inv_e = 1.0 / N_EDGES
    s_sum = s0_ref[...] + s1_ref[...]
    m0 = s_sum[0:1, :] * inv_e
    m2 = s_sum[1:2, :] * inv_e
    var = m2 - m0 * m0
    rstd = lax.rsqrt(var + EPS)
    scale = gam_ref[...] * rstd
    shift = bet_ref[...] - (m0 + b_ref[...]) * scale
    dc, df = _unpack_bf16_pair(gd_ref[...])
    sc_, sf = _unpack_bf16_pair(gs_ref[...])
    y = (jnp.concatenate([dc + sc_, df + sf], axis=1)
         + jnp.dot(ef_ref[...], we_ref[...],
                   preferred_element_type=jnp.float32))
    yn = (y + b_ref[...]) * scale + shift
    sig = yn[:, :NODE_DIM]
    gat = yn[:, NODE_DIM:]
    signal = jnp.maximum(sig, 0.0) + jnp.log(1.0 + jnp.exp(-jnp.abs(sig)))
    gate = 1.0 / (1.0 + jnp.exp(-gat))
    msg_ref[...] = gate * signal


def _apply(s0, s1, bias, gam, bet, gd, gs, ef, We):
    return pl.pallas_call(
        _apply_body,
        grid=(N_EDGE_BLKS,),
        in_specs=[
            pl.BlockSpec((2, PAIR_DIM), lambda i: (0, 0)),
            pl.BlockSpec((2, PAIR_DIM), lambda i: (0, 0)),
            pl.BlockSpec((1, PAIR_DIM), lambda i: (0, 0)),
            pl.BlockSpec((1, PAIR_DIM), lambda i: (0, 0)),
            pl.BlockSpec((1, PAIR_DIM), lambda i: (0, 0)),
            pl.BlockSpec((EDGE_BLK, NODE_DIM), lambda i: (i, 0)),
            pl.BlockSpec((EDGE_BLK, NODE_DIM), lambda i: (i, 0)),
            pl.BlockSpec((EDGE_BLK, EDGE_DIM), lambda i: (i, 0)),
            pl.BlockSpec((EDGE_DIM, PAIR_DIM), lambda i: (0, 0)),
        ],
        out_specs=pl.BlockSpec((EDGE_BLK, NODE_DIM), lambda i: (i, 0)),
        out_shape=jax.ShapeDtypeStruct((HALF_E, NODE_DIM), jnp.float32),
    )(s0, s1, bias, gam, bet, gd, gs, ef, We)


_STRIPE = 640
_STRIPE_LAST = N_NODES - 15 * _STRIPE


@functools.cache
def _scatter_kernel_fn():
    @functools.partial(
        pl.kernel,
        out_type=jax.ShapeDtypeStruct((NC, N_NODES, NODE_DIM), jnp.float32),
        mesh=_sc_mesh(),
        scratch_types=[
            [pltpu.VMEM((CHUNK,), jnp.int32) for _ in range(SC_SLOTS)],
            pltpu.VMEM((SC_SLOTS, CHUNK, NODE_DIM), jnp.float32),
            pltpu.VMEM_SHARED((N_NODES, NODE_DIM), jnp.float32),
            pltpu.SemaphoreType.DMA,
            pltpu.SemaphoreType.DMA,
        ],
    )
    def _scatter_kernel(msg_hbm, dsti_hbm, zer_hbm, out_hbm, di_v, rows_v,
                        acc_sh, semr, sema):
        c = lax.axis_index("c")
        s = lax.axis_index("s")
        wid = s * NC + c
        base = wid * E_PER_W

        @pl.when(s < NS - 1)
        def _z0():
            pltpu.sync_copy(zer_hbm.at[pl.ds(s * _STRIPE, _STRIPE)],
                            acc_sh.at[pl.ds(s * _STRIPE, _STRIPE)])

        @pl.when(s == NS - 1)
        def _z1():
            pltpu.sync_copy(zer_hbm.at[pl.ds(15 * _STRIPE, _STRIPE_LAST)],
                            acc_sh.at[pl.ds(15 * _STRIPE, _STRIPE_LAST)])

        plsc.subcore_barrier()

        def drain_adds():
            for t in range(SC_SLOTS):
                pltpu.make_async_copy(
                    rows_v.at[t], acc_sh.at[di_v[t]], sema).wait()

        def do_group(g, drain_prev):
            if drain_prev:
                drain_adds()
            descs = []
            for t in range(SC_SLOTS):
                off = base + (g * SC_SLOTS + t) * CHUNK
                descs.append(pltpu.async_copy(
                    dsti_hbm.at[pl.ds(off, CHUNK)], di_v[t], semr))
                descs.append(pltpu.async_copy(
                    msg_hbm.at[pl.ds(off, CHUNK)], rows_v.at[t], semr))
            for d in descs:
                d.wait()
            for t in range(SC_SLOTS):
                pltpu.async_copy(rows_v.at[t], acc_sh.at[di_v[t]], sema,
                                 add=True)

        do_group(0, False)

        def body(g, carry):
            do_group(g, True)
            return carry

        lax.fori_loop(1, N_SC_GROUPS, body, 0)
        drain_adds()

        tail_off = base + (N_CHUNKS - 1) * CHUNK
        pltpu.sync_copy(dsti_hbm.at[pl.ds(tail_off, CHUNK)], di_v[0])
        pltpu.sync_copy(msg_hbm.at[pl.ds(tail_off, CHUNK)], rows_v.at[0])
        pltpu.sync_copy(rows_v.at[0], acc_sh.at[di_v[0]], add=True)
        plsc.subcore_barrier()

        @pl.when(s < NS - 1)
        def _w0():
            pltpu.sync_copy(acc_sh.at[pl.ds(s * _STRIPE, _STRIPE)],
                            out_hbm.at[c].at[pl.ds(s * _STRIPE, _STRIPE)])

        @pl.when(s == NS - 1)
        def _w1():
            pltpu.sync_copy(acc_sh.at[pl.ds(15 * _STRIPE, _STRIPE_LAST)],
                            out_hbm.at[c].at[pl.ds(15 * _STRIPE, _STRIPE_LAST)])

    return _scatter_kernel


def _final_body(a_ref, p0_ref, p1_ref, o_ref):
    o_ref[...] = a_ref[...] + p0_ref[0] + p0_ref[1] + p1_ref[0] + p1_ref[1]


def _final(atom, part0, part1):
    return pl.pallas_call(
        _final_body,
        grid=(N_NODE_BLKS,),
        in_specs=[
            pl.BlockSpec((NODE_BLK, NODE_DIM), lambda i: (i, 0)),
            pl.BlockSpec((2, NODE_BLK, NODE_DIM), lambda i: (0, i, 0)),
            pl.BlockSpec((2, NODE_BLK, NODE_DIM), lambda i: (0, i, 0)),
        ],
        out_specs=pl.BlockSpec((NODE_BLK, NODE_DIM), lambda i: (i, 0)),
        out_shape=jax.ShapeDtypeStruct((N_NODES, NODE_DIM), jnp.float32),
    )(atom, part0, part1)


def kernel(atom_features, edge_features, edge_indices, W_filter, b_filter,
           gamma_filter, beta_filter, W_core, b_core, gamma_core, beta_core):
    src_idx = edge_indices[:, 0]
    dst_idx = edge_indices[:, 1]

    Wd = jnp.concatenate([W_core[:NODE_DIM], W_filter[:NODE_DIM]], axis=1)
    Ws = jnp.concatenate([W_core[NODE_DIM:2 * NODE_DIM],
                          W_filter[NODE_DIM:2 * NODE_DIM]], axis=1)
    We = jnp.concatenate([W_core[2 * NODE_DIM:], W_filter[2 * NODE_DIM:]],
                         axis=1)
    bias = jnp.concatenate([b_core, b_filter])[None, :]
    gam = jnp.concatenate([gamma_core, gamma_filter])[None, :]
    bet = jnp.concatenate([beta_core, beta_filter])[None, :]

    pd, ps = _project(atom_features, Wd, Ws)

    dst0, dst1 = dst_idx[:HALF_E], dst_idx[HALF_E:]
    src0, src1 = src_idx[:HALF_E], src_idx[HALF_E:]
    ef0, ef1 = edge_features[:HALF_E], edge_features[HALF_E:]

    gather = _gather_kernel_fn()
    gd0, gs0 = gather(pd, ps, dst0, src0)
    gd1, gs1 = gather(pd, ps, dst1, src1)

    s0 = _stats(gd0, gs0, ef0, We)
    s1 = _stats(gd1, gs1, ef1, We)

    msg0 = _apply(s0, s1, bias, gam, bet, gd0, gs0, ef0, We)
    msg1 = _apply(s0, s1, bias, gam, bet, gd1, gs1, ef1, We)

    zeros = jnp.zeros((N_NODES, NODE_DIM), jnp.float32)
    scatter = _scatter_kernel_fn()
    part0 = scatter(msg0, dst0, zeros)
    part1 = scatter(msg1, dst1, zeros)
    return _final(atom_features, part0, part1)


# speedup vs baseline: 1.1668x; 1.1526x over previous
import functools

import jax
import jax.numpy as jnp
from jax import lax
from jax.experimental import pallas as pl
from jax.experimental.pallas import tpu as pltpu
from jax.experimental.pallas import tpu_sc as plsc

N_NODES = 10000
N_EDGES = 320000
NODE_DIM = 128
EDGE_DIM = 16
PAIR_DIM = 2 * NODE_DIM
EPS = 1e-5

HALF_E = N_EDGES // 2

NC = 2
NS = 16
NW = NC * NS
E_PER_W = HALF_E // NW
CHUNK = 40
N_CHUNKS = E_PER_W // CHUNK

E_SUB = HALF_E // NS
SLOTS = 5
N_GCHUNKS = E_SUB // CHUNK
N_GROUPS = N_GCHUNKS // SLOTS
SC_SLOTS = 4
N_SC_GROUPS = (N_CHUNKS - 1) // SC_SLOTS

EDGE_BLK = 8000
N_EDGE_BLKS = HALF_E // EDGE_BLK
NODE_BLK = 2000
N_NODE_BLKS = N_NODES // NODE_BLK


@functools.cache
def _sc_mesh():
    return plsc.VectorSubcoreMesh(core_axis_name="c", subcore_axis_name="s",
                                  num_cores=NC, num_subcores=NS)


def _pack_bf16_pair(core_f32, filt_f32):
    cb = lax.bitcast_convert_type(core_f32.astype(jnp.bfloat16),
                                  jnp.uint16).astype(jnp.int32)
    fb = lax.bitcast_convert_type(filt_f32.astype(jnp.bfloat16),
                                  jnp.uint16).astype(jnp.int32)
    return cb | (fb << 16)


def _unpack_bf16_pair(packed_i32):
    core = lax.bitcast_convert_type(packed_i32 << 16, jnp.float32)
    filt = lax.bitcast_convert_type((packed_i32 >> 16) << 16, jnp.float32)
    return core, filt


def _proj_body(a_ref, wd_ref, ws_ref, pd_ref, ps_ref):
    a = a_ref[...]
    pd = jnp.dot(a, wd_ref[...], preferred_element_type=jnp.float32)
    ps = jnp.dot(a, ws_ref[...], preferred_element_type=jnp.float32)
    pd_ref[...] = _pack_bf16_pair(pd[:, :NODE_DIM], pd[:, NODE_DIM:])
    ps_ref[...] = _pack_bf16_pair(ps[:, :NODE_DIM], ps[:, NODE_DIM:])


def _project(atom, Wd, Ws):
    return pl.pallas_call(
        _proj_body,
        grid=(N_NODE_BLKS,),
        in_specs=[
            pl.BlockSpec((NODE_BLK, NODE_DIM), lambda i: (i, 0)),
            pl.BlockSpec((NODE_DIM, PAIR_DIM), lambda i: (0, 0)),
            pl.BlockSpec((NODE_DIM, PAIR_DIM), lambda i: (0, 0)),
        ],
        out_specs=[
            pl.BlockSpec((NODE_BLK, NODE_DIM), lambda i: (i, 0)),
            pl.BlockSpec((NODE_BLK, NODE_DIM), lambda i: (i, 0)),
        ],
        out_shape=[
            jax.ShapeDtypeStruct((N_NODES, NODE_DIM), jnp.int32),
            jax.ShapeDtypeStruct((N_NODES, NODE_DIM), jnp.int32),
        ],
    )(atom, Wd, Ws)


@functools.cache
def _gather_kernel_fn():
    @functools.partial(
        pl.kernel,
        out_type=[
            jax.ShapeDtypeStruct((HALF_E, NODE_DIM), jnp.int32),
            jax.ShapeDtypeStruct((HALF_E, NODE_DIM), jnp.int32),
        ],
        mesh=_sc_mesh(),
        scratch_types=[
            pltpu.VMEM((E_SUB,), jnp.int32),
            pltpu.VMEM((SLOTS, CHUNK, NODE_DIM), jnp.int32),
            pltpu.VMEM_SHARED((N_NODES, NODE_DIM), jnp.int32),
            pltpu.SemaphoreType.DMA,
            pltpu.SemaphoreType.DMA,
        ],
    )
    def _gather_kernel(pd_hbm, ps_hbm, dsti_hbm, srci_hbm, gd_hbm, gs_hbm,
                       idx_v, rows_v, tab_sh, semg, semw):
        c = lax.axis_index("c")
        s = lax.axis_index("s")
        base = s * E_SUB

        def run(tab_hbm, idx_hbm, out_hbm):
            @pl.when(s < NS - 1)
            def _t0():
                pltpu.sync_copy(tab_hbm.at[pl.ds(s * _STRIPE, _STRIPE)],
                                tab_sh.at[pl.ds(s * _STRIPE, _STRIPE)])

            @pl.when(s == NS - 1)
            def _t1():
                pltpu.sync_copy(
                    tab_hbm.at[pl.ds(15 * _STRIPE, _STRIPE_LAST)],
                    tab_sh.at[pl.ds(15 * _STRIPE, _STRIPE_LAST)])

            pltpu.sync_copy(idx_hbm.at[pl.ds(base, E_SUB)], idx_v)
            plsc.subcore_barrier()

            def drain_writes():
                for t in range(SLOTS):
                    pltpu.make_async_copy(
                        rows_v.at[t], out_hbm.at[pl.ds(base, CHUNK)],
                        semw).wait()

            def do_group(g, drain_prev):
                if drain_prev:
                    drain_writes()
                descs = []
                for t in range(SLOTS):
                    loc = (g * SLOTS + t) * CHUNK
                    descs.append(pltpu.async_copy(
                        tab_sh.at[idx_v.at[pl.ds(loc, CHUNK)]], rows_v.at[t],
                        semg))
                for d in descs:
                    d.wait()
                for t in range(SLOTS):
                    off = base + (g * SLOTS + t) * CHUNK
                    pltpu.async_copy(rows_v.at[t],
                                     out_hbm.at[pl.ds(off, CHUNK)], semw)

            do_group(0, False)

            def body(g, carry):
                do_group(g, True)
                return carry

            lax.fori_loop(1, N_GROUPS, body, 0)
            drain_writes()

        @pl.when(c == 0)
        def _dst():
            run(pd_hbm, dsti_hbm, gd_hbm)

        @pl.when(c == 1)
        def _src():
            run(ps_hbm, srci_hbm, gs_hbm)

    return _gather_kernel


def _stats_body(gd_ref, gs_ref, ef_ref, we_ref, out_ref, acc_ref):
    i = pl.program_id(0)

    @pl.when(i == 0)
    def _init():
        acc_ref[...] = jnp.zeros_like(acc_ref)

    dc, df = _unpack_bf16_pair(gd_ref[...])
    sc_, sf = _unpack_bf16_pair(gs_ref[...])
    y = (jnp.concatenate([dc + sc_, df + sf], axis=1)
         + jnp.dot(ef_ref[...], we_ref[...],
                   preferred_element_type=jnp.float32))
    acc_ref[0:1, :] += jnp.sum(y, axis=0, keepdims=True)
    acc_ref[1:2, :] += jnp.sum(y * y, axis=0, keepdims=True)

    @pl.when(i == pl.num_programs(0) - 1)
    def _fin():
        out_ref[...] = acc_ref[...]


def _stats(gd, gs, ef, We):
    return pl.pallas_call(
        _stats_body,
        grid=(N_EDGE_BLKS,),
        in_specs=[
            pl.BlockSpec((EDGE_BLK, NODE_DIM), lambda i: (i, 0)),
            pl.BlockSpec((EDGE_BLK, NODE_DIM), lambda i: (i, 0)),
            pl.BlockSpec((EDGE_BLK, EDGE_DIM), lambda i: (i, 0)),
            pl.BlockSpec((EDGE_DIM, PAIR_DIM), lambda i: (0, 0)),
        ],
        out_specs=pl.BlockSpec((2, PAIR_DIM), lambda i: (0, 0)),
        out_shape=jax.ShapeDtypeStruct((2, PAIR_DIM), jnp.float32),
        scratch_shapes=[pltpu.VMEM((2, PAIR_DIM), jnp.float32)],
    )(gd, gs, ef, We)


def _apply_body(s0_ref, s1_ref, b_ref, gam_ref, bet_ref, gd_ref, gs_ref,
                ef_ref, we_ref, msg_ref):
    inv_e = 1.0 / N_EDGES
    s_sum = s0_ref[...] + s1_ref[...]
    m0 = s_sum[0:1, :] * inv_e
    m2 = s_sum[1:2, :] * inv_e
    var = m2 - m0 * m0
    rstd = lax.rsqrt(var + EPS)
    scale = gam_ref[...] * rstd
    shift = bet_ref[...] - (m0 + b_ref[...]) * scale
    dc, df = _unpack_bf16_pair(gd_ref[...])
    sc_, sf = _unpack_bf16_pair(gs_ref[...])
    y = (jnp.concatenate([dc + sc_, df + sf], axis=1)
         + jnp.dot(ef_ref[...], we_ref[...],
                   preferred_element_type=jnp.float32))
    yn = (y + b_ref[...]) * scale + shift
    sig = yn[:, :NODE_DIM]
    gat = yn[:, NODE_DIM:]
    signal = jnp.maximum(sig, 0.0) + jnp.log(1.0 + jnp.exp(-jnp.abs(sig)))
    gate = 1.0 / (1.0 + jnp.exp(-gat))
    msg_ref[...] = gate * signal


def _apply(s0, s1, bias, gam, bet, gd, gs, ef, We):
    return pl.pallas_call(
        _apply_body,
        grid=(N_EDGE_BLKS,),
        in_specs=[
            pl.BlockSpec((2, PAIR_DIM), lambda i: (0, 0)),
            pl.BlockSpec((2, PAIR_DIM), lambda i: (0, 0)),
            pl.BlockSpec((1, PAIR_DIM), lambda i: (0, 0)),
            pl.BlockSpec((1, PAIR_DIM), lambda i: (0, 0)),
            pl.BlockSpec((1, PAIR_DIM), lambda i: (0, 0)),
            pl.BlockSpec((EDGE_BLK, NODE_DIM), lambda i: (i, 0)),
            pl.BlockSpec((EDGE_BLK, NODE_DIM), lambda i: (i, 0)),
            pl.BlockSpec((EDGE_BLK, EDGE_DIM), lambda i: (i, 0)),
            pl.BlockSpec((EDGE_DIM, PAIR_DIM), lambda i: (0, 0)),
        ],
        out_specs=pl.BlockSpec((EDGE_BLK, NODE_DIM), lambda i: (i, 0)),
        out_shape=jax.ShapeDtypeStruct((HALF_E, NODE_DIM), jnp.float32),
    )(s0, s1, bias, gam, bet, gd, gs, ef, We)


_STRIPE = 640
_STRIPE_LAST = N_NODES - 15 * _STRIPE


@functools.cache
def _scatter_kernel_fn():
    @functools.partial(
        pl.kernel,
        out_type=jax.ShapeDtypeStruct((NC, N_NODES, NODE_DIM), jnp.float32),
        mesh=_sc_mesh(),
        scratch_types=[
            [pltpu.VMEM((CHUNK,), jnp.int32) for _ in range(SC_SLOTS)],
            pltpu.VMEM((SC_SLOTS, CHUNK, NODE_DIM), jnp.float32),
            pltpu.VMEM_SHARED((N_NODES, NODE_DIM), jnp.float32),
            pltpu.SemaphoreType.DMA,
            pltpu.SemaphoreType.DMA,
        ],
    )
    def _scatter_kernel(msg_hbm, dsti_hbm, zer_hbm, out_hbm, di_v, rows_v,
                        acc_sh, semr, sema):
        c = lax.axis_index("c")
        s = lax.axis_index("s")
        wid = s * NC + c
        base = wid * E_PER_W

        @pl.when(s < NS - 1)
        def _z0():
            pltpu.sync_copy(zer_hbm.at[pl.ds(s * _STRIPE, _STRIPE)],
                            acc_sh.at[pl.ds(s * _STRIPE, _STRIPE)])

        @pl.when(s == NS - 1)
        def _z1():
            pltpu.sync_copy(zer_hbm.at[pl.ds(15 * _STRIPE, _STRIPE_LAST)],
                            acc_sh.at[pl.ds(15 * _STRIPE, _STRIPE_LAST)])

        plsc.subcore_barrier()

        def drain_adds():
            for t in range(SC_SLOTS):
                pltpu.make_async_copy(
                    rows_v.at[t], acc_sh.at[di_v[t]], sema).wait()

        def do_group(g, drain_prev):
            if drain_prev:
                drain_adds()
            descs = []
            for t in range(SC_SLOTS):
                off = base + (g * SC_SLOTS + t) * CHUNK
                descs.append(pltpu.async_copy(
                    dsti_hbm.at[pl.ds(off, CHUNK)], di_v[t], semr))
                descs.append(pltpu.async_copy(
                    msg_hbm.at[pl.ds(off, CHUNK)], rows_v.at[t], semr))
            for d in descs:
                d.wait()
            for t in range(SC_SLOTS):
                pltpu.async_copy(rows_v.at[t], acc_sh.at[di_v[t]], sema,
                                 add=True)

        do_group(0, False)

        def body(g, carry):
            do_group(g, True)
            return carry

        lax.fori_loop(1, N_SC_GROUPS, body, 0)
        drain_adds()

        tail_off = base + (N_CHUNKS - 1) * CHUNK
        pltpu.sync_copy(dsti_hbm.at[pl.ds(tail_off, CHUNK)], di_v[0])
        pltpu.sync_copy(msg_hbm.at[pl.ds(tail_off, CHUNK)], rows_v.at[0])
        pltpu.sync_copy(rows_v.at[0], acc_sh.at[di_v[0]], add=True)
        plsc.subcore_barrier()

        @pl.when(s < NS - 1)
        def _w0():
            pltpu.sync_copy(acc_sh.at[pl.ds(s * _STRIPE, _STRIPE)],
                            out_hbm.at[c].at[pl.ds(s * _STRIPE, _STRIPE)])

        @pl.when(s == NS - 1)
        def _w1():
            pltpu.sync_copy(acc_sh.at[pl.ds(15 * _STRIPE, _STRIPE_LAST)],
                            out_hbm.at[c].at[pl.ds(15 * _STRIPE, _STRIPE_LAST)])

    return _scatter_kernel


def _final_body(a_ref, p0_ref, p1_ref, o_ref):
    o_ref[...] = a_ref[...] + p0_ref[0] + p0_ref[1] + p1_ref[0] + p1_ref[1]


def _final(atom, part0, part1):
    return pl.pallas_call(
        _final_body,
        grid=(N_NODE_BLKS,),
        in_specs=[
            pl.BlockSpec((NODE_BLK, NODE_DIM), lambda i: (i, 0)),
            pl.BlockSpec((2, NODE_BLK, NODE_DIM), lambda i: (0, i, 0)),
            pl.BlockSpec((2, NODE_BLK, NODE_DIM), lambda i: (0, i, 0)),
        ],
        out_specs=pl.BlockSpec((NODE_BLK, NODE_DIM), lambda i: (i, 0)),
        out_shape=jax.ShapeDtypeStruct((N_NODES, NODE_DIM), jnp.float32),
    )(atom, part0, part1)


def kernel(atom_features, edge_features, edge_indices, W_filter, b_filter,
           gamma_filter, beta_filter, W_core, b_core, gamma_core, beta_core):
    src_idx = edge_indices[:, 0]
    dst_idx = edge_indices[:, 1]

    Wd = jnp.concatenate([W_core[:NODE_DIM], W_filter[:NODE_DIM]], axis=1)
    Ws = jnp.concatenate([W_core[NODE_DIM:2 * NODE_DIM],
                          W_filter[NODE_DIM:2 * NODE_DIM]], axis=1)
    We = jnp.concatenate([W_core[2 * NODE_DIM:], W_filter[2 * NODE_DIM:]],
                         axis=1)
    bias = jnp.concatenate([b_core, b_filter])[None, :]
    gam = jnp.concatenate([gamma_core, gamma_filter])[None, :]
    bet = jnp.concatenate([beta_core, beta_filter])[None, :]

    pd, ps = _project(atom_features, Wd, Ws)

    dst0, dst1 = dst_idx[:HALF_E], dst_idx[HALF_E:]
    src0, src1 = src_idx[:HALF_E], src_idx[HALF_E:]
    ef0, ef1 = edge_features[:HALF_E], edge_features[HALF_E:]

    gather = _gather_kernel_fn()
    gd0, gs0 = gather(pd, ps, dst0, src0)
    gd1, gs1 = gather(pd, ps, dst1, src1)

    s0 = _stats(gd0, gs0, ef0, We)
    s1 = _stats(gd1, gs1, ef1, We)

    msg0 = _apply(s0, s1, bias, gam, bet, gd0, gs0, ef0, We)
    msg1 = _apply(s0, s1, bias, gam, bet, gd1, gs1, ef1, We)

    zeros = jnp.zeros((N_NODES, NODE_DIM), jnp.float32)
    scatter = _scatter_kernel_fn()
    part0 = scatter(msg0, dst0, zeros)
    part1 = scatter(msg1, dst1, zeros)
    return _final(atom_features, part0, part1)


# speedup vs baseline: 1.1985x; 1.0272x over previous
import functools

import jax
import jax.numpy as jnp
from jax import lax
from jax.experimental import pallas as pl
from jax.experimental.pallas import tpu as pltpu
from jax.experimental.pallas import tpu_sc as plsc

N_NODES = 10000
N_EDGES = 320000
NODE_DIM = 128
EDGE_DIM = 16
PAIR_DIM = 2 * NODE_DIM
EPS = 1e-5

HALF_E = N_EDGES // 2

NC = 2
NS = 16
NW = NC * NS
E_PER_W = HALF_E // NW
CHUNK = 40
N_CHUNKS = E_PER_W // CHUNK

E_SUB = HALF_E // NS
SLOTS = 5
N_GCHUNKS = E_SUB // CHUNK
N_GROUPS = N_GCHUNKS // SLOTS
SC_SLOTS = 5
N_SC_GROUPS = N_CHUNKS // SC_SLOTS

EDGE_BLK = 8000
N_EDGE_BLKS = HALF_E // EDGE_BLK
NODE_BLK = 2000
N_NODE_BLKS = N_NODES // NODE_BLK


@functools.cache
def _sc_mesh():
    return plsc.VectorSubcoreMesh(core_axis_name="c", subcore_axis_name="s",
                                  num_cores=NC, num_subcores=NS)


def _pack_bf16_pair(core_f32, filt_f32):
    cb = lax.bitcast_convert_type(core_f32.astype(jnp.bfloat16),
                                  jnp.uint16).astype(jnp.int32)
    fb = lax.bitcast_convert_type(filt_f32.astype(jnp.bfloat16),
                                  jnp.uint16).astype(jnp.int32)
    return cb | (fb << 16)


def _unpack_bf16_pair(packed_i32):
    core = lax.bitcast_convert_type(packed_i32 << 16, jnp.float32)
    filt = lax.bitcast_convert_type((packed_i32 >> 16) << 16, jnp.float32)
    return core, filt


def _proj_body(a_ref, wd_ref, ws_ref, pd_ref, ps_ref):
    a = a_ref[...]
    pd = jnp.dot(a, wd_ref[...], preferred_element_type=jnp.float32)
    ps = jnp.dot(a, ws_ref[...], preferred_element_type=jnp.float32)
    pd_ref[...] = _pack_bf16_pair(pd[:, :NODE_DIM], pd[:, NODE_DIM:])
    ps_ref[...] = _pack_bf16_pair(ps[:, :NODE_DIM], ps[:, NODE_DIM:])


def _project(atom, Wd, Ws):
    return pl.pallas_call(
        _proj_body,
        grid=(N_NODE_BLKS,),
        in_specs=[
            pl.BlockSpec((NODE_BLK, NODE_DIM), lambda i: (i, 0)),
            pl.BlockSpec((NODE_DIM, PAIR_DIM), lambda i: (0, 0)),
            pl.BlockSpec((NODE_DIM, PAIR_DIM), lambda i: (0, 0)),
        ],
        out_specs=[
            pl.BlockSpec((NODE_BLK, NODE_DIM), lambda i: (i, 0)),
            pl.BlockSpec((NODE_BLK, NODE_DIM), lambda i: (i, 0)),
        ],
        out_shape=[
            jax.ShapeDtypeStruct((N_NODES, NODE_DIM), jnp.int32),
            jax.ShapeDtypeStruct((N_NODES, NODE_DIM), jnp.int32),
        ],
    )(atom, Wd, Ws)


@functools.cache
def _gather_kernel_fn():
    @functools.partial(
        pl.kernel,
        out_type=[
            jax.ShapeDtypeStruct((HALF_E, NODE_DIM), jnp.int32),
            jax.ShapeDtypeStruct((HALF_E, NODE_DIM), jnp.int32),
        ],
        mesh=_sc_mesh(),
        scratch_types=[
            pltpu.VMEM((E_SUB,), jnp.int32),
            pltpu.VMEM((SLOTS, CHUNK, NODE_DIM), jnp.int32),
            pltpu.VMEM_SHARED((N_NODES, NODE_DIM), jnp.int32),
            pltpu.SemaphoreType.DMA,
            pltpu.SemaphoreType.DMA,
        ],
    )
    def _gather_kernel(pd_hbm, ps_hbm, dsti_hbm, srci_hbm, gd_hbm, gs_hbm,
                       idx_v, rows_v, tab_sh, semg, semw):
        c = lax.axis_index("c")
        s = lax.axis_index("s")
        base = s * E_SUB

        def run(tab_hbm, idx_hbm, out_hbm):
            @pl.when(s < NS - 1)
            def _t0():
                pltpu.sync_copy(tab_hbm.at[pl.ds(s * _STRIPE, _STRIPE)],
                                tab_sh.at[pl.ds(s * _STRIPE, _STRIPE)])

            @pl.when(s == NS - 1)
            def _t1():
                pltpu.sync_copy(
                    tab_hbm.at[pl.ds(15 * _STRIPE, _STRIPE_LAST)],
                    tab_sh.at[pl.ds(15 * _STRIPE, _STRIPE_LAST)])

            pltpu.sync_copy(idx_hbm.at[pl.ds(base, E_SUB)], idx_v)
            plsc.subcore_barrier()

            def drain_writes():
                for t in range(SLOTS):
                    pltpu.make_async_copy(
                        rows_v.at[t], out_hbm.at[pl.ds(base, CHUNK)],
                        semw).wait()

            def do_group(g, drain_prev):
                if drain_prev:
                    drain_writes()
                descs = []
                for t in range(SLOTS):
                    loc = (g * SLOTS + t) * CHUNK
                    descs.append(pltpu.async_copy(
                        tab_sh.at[idx_v.at[pl.ds(loc, CHUNK)]], rows_v.at[t],
                        semg))
                for d in descs:
                    d.wait()
                for t in range(SLOTS):
                    off = base + (g * SLOTS + t) * CHUNK
                    pltpu.async_copy(rows_v.at[t],
                                     out_hbm.at[pl.ds(off, CHUNK)], semw)

            do_group(0, False)

            def body(g, carry):
                do_group(g, True)
                return carry

            lax.fori_loop(1, N_GROUPS, body, 0)
            drain_writes()

        @pl.when(c == 0)
        def _dst():
            run(pd_hbm, dsti_hbm, gd_hbm)

        @pl.when(c == 1)
        def _src():
            run(ps_hbm, srci_hbm, gs_hbm)

    return _gather_kernel


def _stats_body(gd_ref, gs_ref, ef_ref, we_ref, out_ref, yp_ref, acc_ref):
    i = pl.program_id(0)

    @pl.when(i == 0)
    def _init():
        acc_ref[...] = jnp.zeros_like(acc_ref)

    dc, df = _unpack_bf16_pair(gd_ref[...])
    sc_, sf = _unpack_bf16_pair(gs_ref[...])
    y = (jnp.concatenate([dc + sc_, df + sf], axis=1)
         + jnp.dot(ef_ref[...], we_ref[...],
                   preferred_element_type=jnp.float32))
    acc_ref[0:1, :] += jnp.sum(y, axis=0, keepdims=True)
    acc_ref[1:2, :] += jnp.sum(y * y, axis=0, keepdims=True)
    yp_ref[...] = _pack_bf16_pair(y[:, :NODE_DIM], y[:, NODE_DIM:])

    @pl.when(i == pl.num_programs(0) - 1)
    def _fin():
        out_ref[...] = acc_ref[...]


def _stats(gd, gs, ef, We):
    return pl.pallas_call(
        _stats_body,
        grid=(N_EDGE_BLKS,),
        in_specs=[
            pl.BlockSpec((EDGE_BLK, NODE_DIM), lambda i: (i, 0)),
            pl.BlockSpec((EDGE_BLK, NODE_DIM), lambda i: (i, 0)),
            pl.BlockSpec((EDGE_BLK, EDGE_DIM), lambda i: (i, 0)),
            pl.BlockSpec((EDGE_DIM, PAIR_DIM), lambda i: (0, 0)),
        ],
        out_specs=[
            pl.BlockSpec((2, PAIR_DIM), lambda i: (0, 0)),
            pl.BlockSpec((EDGE_BLK, NODE_DIM), lambda i: (i, 0)),
        ],
        out_shape=[
            jax.ShapeDtypeStruct((2, PAIR_DIM), jnp.float32),
            jax.ShapeDtypeStruct((HALF_E, NODE_DIM), jnp.int32),
        ],
        scratch_shapes=[pltpu.VMEM((2, PAIR_DIM), jnp.float32)],
    )(gd, gs, ef, We)


def _apply_body(s0_ref, s1_ref, b_ref, gam_ref, bet_ref, yp_ref, msg_ref):
    inv_e = 1.0 / N_EDGES
    s_sum = s0_ref[...] + s1_ref[...]
    m0 = s_sum[0:1, :] * inv_e
    m2 = s_sum[1:2, :] * inv_e
    var = m2 - m0 * m0
    rstd = lax.rsqrt(var + EPS)
    scale = gam_ref[...] * rstd
    shift = bet_ref[...] - m0 * scale
    yc, yf = _unpack_bf16_pair(yp_ref[...])
    sig = yc * scale[:, :NODE_DIM] + shift[:, :NODE_DIM]
    gat = yf * scale[:, NODE_DIM:] + shift[:, NODE_DIM:]
    signal = jnp.maximum(sig, 0.0) + jnp.log(1.0 + jnp.exp(-jnp.abs(sig)))
    gate = 1.0 / (1.0 + jnp.exp(-gat))
    msg_ref[...] = gate * signal


def _apply(s0, s1, bias, gam, bet, yp):
    return pl.pallas_call(
        _apply_body,
        grid=(N_EDGE_BLKS,),
        in_specs=[
            pl.BlockSpec((2, PAIR_DIM), lambda i: (0, 0)),
            pl.BlockSpec((2, PAIR_DIM), lambda i: (0, 0)),
            pl.BlockSpec((1, PAIR_DIM), lambda i: (0, 0)),
            pl.BlockSpec((1, PAIR_DIM), lambda i: (0, 0)),
            pl.BlockSpec((1, PAIR_DIM), lambda i: (0, 0)),
            pl.BlockSpec((EDGE_BLK, NODE_DIM), lambda i: (i, 0)),
        ],
        out_specs=pl.BlockSpec((EDGE_BLK, NODE_DIM), lambda i: (i, 0)),
        out_shape=jax.ShapeDtypeStruct((HALF_E, NODE_DIM), jnp.float32),
    )(s0, s1, bias, gam, bet, yp)


_STRIPE = 640
_STRIPE_LAST = N_NODES - 15 * _STRIPE


@functools.cache
def _scatter_kernel_fn():
    @functools.partial(
        pl.kernel,
        out_type=jax.ShapeDtypeStruct((NC, N_NODES, NODE_DIM), jnp.float32),
        mesh=_sc_mesh(),
        scratch_types=[
            [pltpu.VMEM((CHUNK,), jnp.int32) for _ in range(SC_SLOTS)],
            pltpu.VMEM((SC_SLOTS, CHUNK, NODE_DIM), jnp.float32),
            pltpu.VMEM_SHARED((N_NODES, NODE_DIM), jnp.float32),
            pltpu.SemaphoreType.DMA,
            pltpu.SemaphoreType.DMA,
        ],
    )
    def _scatter_kernel(msg_hbm, dsti_hbm, zer_hbm, out_hbm, di_v, rows_v,
                        acc_sh, semr, sema):
        c = lax.axis_index("c")
        s = lax.axis_index("s")
        wid = s * NC + c
        base = wid * E_PER_W

        @pl.when(s < NS - 1)
        def _z0():
            pltpu.sync_copy(zer_hbm.at[pl.ds(s * _STRIPE, _STRIPE)],
                            acc_sh.at[pl.ds(s * _STRIPE, _STRIPE)])

        @pl.when(s == NS - 1)
        def _z1():
            pltpu.sync_copy(zer_hbm.at[pl.ds(15 * _STRIPE, _STRIPE_LAST)],
                            acc_sh.at[pl.ds(15 * _STRIPE, _STRIPE_LAST)])

        plsc.subcore_barrier()

        def drain_adds():
            for t in range(SC_SLOTS):
                pltpu.make_async_copy(
                    rows_v.at[t], acc_sh.at[di_v[t]], sema).wait()

        def do_group(g, drain_prev):
            if drain_prev:
                drain_adds()
            descs = []
            for t in range(SC_SLOTS):
                off = base + (g * SC_SLOTS + t) * CHUNK
                descs.append(pltpu.async_copy(
                    dsti_hbm.at[pl.ds(off, CHUNK)], di_v[t], semr))
                descs.append(pltpu.async_copy(
                    msg_hbm.at[pl.ds(off, CHUNK)], rows_v.at[t], semr))
            for d in descs:
                d.wait()
            for t in range(SC_SLOTS):
                pltpu.async_copy(rows_v.at[t], acc_sh.at[di_v[t]], sema,
                                 add=True)

        do_group(0, False)

        def body(g, carry):
            do_group(g, True)
            return carry

        lax.fori_loop(1, N_SC_GROUPS, body, 0)
        drain_adds()
        plsc.subcore_barrier()

        @pl.when(s < NS - 1)
        def _w0():
            pltpu.sync_copy(acc_sh.at[pl.ds(s * _STRIPE, _STRIPE)],
                            out_hbm.at[c].at[pl.ds(s * _STRIPE, _STRIPE)])

        @pl.when(s == NS - 1)
        def _w1():
            pltpu.sync_copy(acc_sh.at[pl.ds(15 * _STRIPE, _STRIPE_LAST)],
                            out_hbm.at[c].at[pl.ds(15 * _STRIPE, _STRIPE_LAST)])

    return _scatter_kernel


def _final_body(a_ref, p0_ref, p1_ref, o_ref):
    o_ref[...] = a_ref[...] + p0_ref[0] + p0_ref[1] + p1_ref[0] + p1_ref[1]


def _final(atom, part0, part1):
    return pl.pallas_call(
        _final_body,
        grid=(N_NODE_BLKS,),
        in_specs=[
            pl.BlockSpec((NODE_BLK, NODE_DIM), lambda i: (i, 0)),
            pl.BlockSpec((2, NODE_BLK, NODE_DIM), lambda i: (0, i, 0)),
            pl.BlockSpec((2, NODE_BLK, NODE_DIM), lambda i: (0, i, 0)),
        ],
        out_specs=pl.BlockSpec((NODE_BLK, NODE_DIM), lambda i: (i, 0)),
        out_shape=jax.ShapeDtypeStruct((N_NODES, NODE_DIM), jnp.float32),
    )(atom, part0, part1)


def kernel(atom_features, edge_features, edge_indices, W_filter, b_filter,
           gamma_filter, beta_filter, W_core, b_core, gamma_core, beta_core):
    src_idx = edge_indices[:, 0]
    dst_idx = edge_indices[:, 1]

    Wd = jnp.concatenate([W_core[:NODE_DIM], W_filter[:NODE_DIM]], axis=1)
    Ws = jnp.concatenate([W_core[NODE_DIM:2 * NODE_DIM],
                          W_filter[NODE_DIM:2 * NODE_DIM]], axis=1)
    We = jnp.concatenate([W_core[2 * NODE_DIM:], W_filter[2 * NODE_DIM:]],
                         axis=1)
    bias = jnp.concatenate([b_core, b_filter])[None, :]
    gam = jnp.concatenate([gamma_core, gamma_filter])[None, :]
    bet = jnp.concatenate([beta_core, beta_filter])[None, :]

    pd, ps = _project(atom_features, Wd, Ws)

    dst0, dst1 = dst_idx[:HALF_E], dst_idx[HALF_E:]
    src0, src1 = src_idx[:HALF_E], src_idx[HALF_E:]
    ef0, ef1 = edge_features[:HALF_E], edge_features[HALF_E:]

    gather = _gather_kernel_fn()
    gd0, gs0 = gather(pd, ps, dst0, src0)
    gd1, gs1 = gather(pd, ps, dst1, src1)

    s0, yp0 = _stats(gd0, gs0, ef0, We)
    s1, yp1 = _stats(gd1, gs1, ef1, We)

    msg0 = _apply(s0, s1, bias, gam, bet, yp0)
    msg1 = _apply(s0, s1, bias, gam, bet, yp1)

    zeros = jnp.zeros((N_NODES, NODE_DIM), jnp.float32)
    scatter = _scatter_kernel_fn()
    part0 = scatter(msg0, dst0, zeros)
    part1 = scatter(msg1, dst1, zeros)
    return _final(atom_features, part0, part1)
